# baseline (speedup 1.0000x reference)
import numpy as np
import jax
import jax.numpy as jnp
from jax.experimental import pallas as pl
from jax.experimental.pallas import tpu as pltpu

_ALPHA = 0.99
_EPS = 1e-14
_LC = 256


def _scan_mats(L: int):
    t = np.arange(L, dtype=np.float64)
    U = np.where(
        t[:, None] <= t[None, :],
        (1.0 - _ALPHA) * _ALPHA ** (t[None, :] - t[:, None]),
        0.0,
    )
    a_row = _ALPHA ** (t + 1.0)
    return U.astype(np.float32), a_row.reshape(1, L).astype(np.float32)


def _eun_kernel(x_ref, u_ref, a_ref, s0_ref, o_ref):
    fdim, pdim, t = x_ref.shape[1], x_ref.shape[2], x_ref.shape[3]
    f2 = fdim * pdim
    val3 = x_ref[0]
    v2 = val3 * val3
    ps3 = v2 + jnp.concatenate([v2[:, 1:2, :], v2[:, 0:1, :]], axis=1)
    m = jnp.sqrt(ps3.reshape(f2, t) + _EPS)
    val = val3.reshape(f2, t)
    carry = s0_ref[...]
    u_full = u_ref[...]
    a_full = a_ref[...]
    o = 0
    while o < t:
        L = min(_LC, t - o)
        m_c = m[:, o : o + L]
        u = u_full[:L, :L]
        a = a_full[:, :L]
        states = (
            jnp.dot(m_c, u, preferred_element_type=jnp.float32) + carry * a
        )
        carry = states[:, L - 1 : L]
        r = jax.lax.rsqrt(states)
        o_ref[0, :, :, o : o + L] = (val[:, o : o + L] * r).reshape(fdim, pdim, L)
        o += L


def kernel(x, init_state):
    b, c, t, f, p = x.shape
    bc = b * c
    xt = jnp.transpose(x, (0, 1, 3, 4, 2)).reshape(bc, f, p, t)
    s0 = jnp.repeat(init_state.reshape(f), p).reshape(f * p, 1)
    u_np, a_np = _scan_mats(_LC)

    out = pl.pallas_call(
        _eun_kernel,
        out_shape=jax.ShapeDtypeStruct((bc, f, p, t), x.dtype),
        grid=(bc,),
        in_specs=[
            pl.BlockSpec((1, f, p, t), lambda i: (i, 0, 0, 0)),
            pl.BlockSpec((_LC, _LC), lambda i: (0, 0)),
            pl.BlockSpec((1, _LC), lambda i: (0, 0)),
            pl.BlockSpec((f * p, 1), lambda i: (0, 0)),
        ],
        out_specs=pl.BlockSpec((1, f, p, t), lambda i: (i, 0, 0, 0)),
        compiler_params=pltpu.CompilerParams(
            dimension_semantics=("arbitrary",),
        ),
        name="exp_unit_norm",
    )(xt, jnp.asarray(u_np), jnp.asarray(a_np), s0)

    return jnp.transpose(out.reshape(b, c, f, p, t), (0, 1, 4, 2, 3))


# speedup vs baseline: 1.0460x; 1.0460x over previous
import numpy as np
import jax
import jax.numpy as jnp
from jax.experimental import pallas as pl
from jax.experimental.pallas import tpu as pltpu

_ALPHA = 0.99
_EPS = 1e-14
_LC = 256


def _scan_mats(L: int):
    t = np.arange(L, dtype=np.float64)
    U = np.where(
        t[:, None] <= t[None, :],
        (1.0 - _ALPHA) * _ALPHA ** (t[None, :] - t[:, None]),
        0.0,
    )
    a_row = _ALPHA ** (t + 1.0)
    return U.astype(np.float32), a_row.reshape(1, L).astype(np.float32)


def _eun_kernel(x_ref, u_ref, a_ref, s0_ref, o_ref):
    val = x_ref[0]
    v2 = val * val
    mag2 = v2[:, 0, :] + v2[:, 1, :]
    m = jnp.sqrt(mag2 + _EPS)
    carry = s0_ref[...]
    u_full = u_ref[...]
    a_full = a_ref[...]
    t = m.shape[1]
    chunks = []
    o = 0
    while o < t:
        L = min(_LC, t - o)
        m_c = m[:, o : o + L]
        u = u_full[:L, :L]
        a = a_full[:, :L]
        states = (
            jnp.dot(m_c, u, preferred_element_type=jnp.float32) + carry * a
        )
        carry = states[:, L - 1 : L]
        chunks.append(states)
        o += L
    states_full = jnp.concatenate(chunks, axis=1)
    r = jax.lax.rsqrt(states_full)
    r_pair = jnp.broadcast_to(r[:, None, :], (r.shape[0], 2, t))
    o_ref[0] = val * r_pair


def kernel(x, init_state):
    b, c, t, f, p = x.shape
    bc = b * c
    xt = jnp.transpose(x, (0, 1, 3, 4, 2)).reshape(bc, f, p, t)
    s0 = init_state.reshape(f, 1)
    u_np, a_np = _scan_mats(_LC)

    out = pl.pallas_call(
        _eun_kernel,
        out_shape=jax.ShapeDtypeStruct((bc, f, p, t), x.dtype),
        grid=(bc,),
        in_specs=[
            pl.BlockSpec((1, f, p, t), lambda i: (i, 0, 0, 0)),
            pl.BlockSpec((_LC, _LC), lambda i: (0, 0)),
            pl.BlockSpec((1, _LC), lambda i: (0, 0)),
            pl.BlockSpec((f, 1), lambda i: (0, 0)),
        ],
        out_specs=pl.BlockSpec((1, f, p, t), lambda i: (i, 0, 0, 0)),
        compiler_params=pltpu.CompilerParams(
            dimension_semantics=("arbitrary",),
        ),
        name="exp_unit_norm",
    )(xt, jnp.asarray(u_np), jnp.asarray(a_np), s0)

    return jnp.transpose(out.reshape(b, c, f, p, t), (0, 1, 4, 2, 3))


# speedup vs baseline: 1.8624x; 1.7805x over previous
import numpy as np
import jax
import jax.numpy as jnp
from jax.experimental import pallas as pl
from jax.experimental.pallas import tpu as pltpu

_ALPHA = 0.99
_EPS = 1e-14
_LC = 256


def _scan_mats(L: int):
    t = np.arange(L, dtype=np.float64)
    U = np.where(
        t[:, None] <= t[None, :],
        (1.0 - _ALPHA) * _ALPHA ** (t[None, :] - t[:, None]),
        0.0,
    )
    a_row = _ALPHA ** (t + 1.0)
    return U.astype(np.float32), a_row.reshape(1, L).astype(np.float32)


def _eun_kernel(x_ref, u_ref, a_ref, s0_ref, o_ref):
    re = x_ref[0, :, 0, :]
    im = x_ref[0, :, 1, :]
    mag2 = re * re + im * im
    m = jnp.sqrt(mag2 + _EPS)
    carry = s0_ref[...]
    u_full = u_ref[...]
    a_full = a_ref[...]
    t = m.shape[1]
    chunks = []
    o = 0
    while o < t:
        L = min(_LC, t - o)
        m_c = m[:, o : o + L]
        u = u_full[:L, :L]
        a = a_full[:, :L]
        states = (
            jnp.dot(m_c, u, preferred_element_type=jnp.float32) + carry * a
        )
        carry = states[:, L - 1 : L]
        chunks.append(states)
        o += L
    states_full = jnp.concatenate(chunks, axis=1)
    r = jax.lax.rsqrt(states_full)
    o_ref[0, :, 0, :] = re * r
    o_ref[0, :, 1, :] = im * r


def kernel(x, init_state):
    b, c, t, f, p = x.shape
    bc = b * c
    xt = jnp.transpose(x, (0, 1, 3, 4, 2)).reshape(bc, f, p, t)
    s0 = init_state.reshape(f, 1)
    u_np, a_np = _scan_mats(_LC)

    out = pl.pallas_call(
        _eun_kernel,
        out_shape=jax.ShapeDtypeStruct((bc, f, p, t), x.dtype),
        grid=(bc,),
        in_specs=[
            pl.BlockSpec((1, f, p, t), lambda i: (i, 0, 0, 0)),
            pl.BlockSpec((_LC, _LC), lambda i: (0, 0)),
            pl.BlockSpec((1, _LC), lambda i: (0, 0)),
            pl.BlockSpec((f, 1), lambda i: (0, 0)),
        ],
        out_specs=pl.BlockSpec((1, f, p, t), lambda i: (i, 0, 0, 0)),
        compiler_params=pltpu.CompilerParams(
            dimension_semantics=("arbitrary",),
        ),
        name="exp_unit_norm",
    )(xt, jnp.asarray(u_np), jnp.asarray(a_np), s0)

    return jnp.transpose(out.reshape(b, c, f, p, t), (0, 1, 4, 2, 3))


# speedup vs baseline: 2.0106x; 1.0796x over previous
import numpy as np
import jax
import jax.numpy as jnp
from jax.experimental import pallas as pl
from jax.experimental.pallas import tpu as pltpu

_ALPHA = 0.99
_EPS = 1e-14
_LC = 256


def _scan_mats(L: int):
    t = np.arange(L, dtype=np.float64)
    U = np.where(
        t[:, None] <= t[None, :],
        (1.0 - _ALPHA) * _ALPHA ** (t[None, :] - t[:, None]),
        0.0,
    )
    a_row = _ALPHA ** (t + 1.0)
    return U.astype(np.float32), a_row.reshape(1, L).astype(np.float32)


def _eun_kernel(x_ref, u_ref, a_ref, s0_ref, o_ref):
    u_full = u_ref[...]
    a_full = a_ref[...]
    t = x_ref.shape[3]
    for g in range(x_ref.shape[0]):
        re = x_ref[g, :, 0, :]
        im = x_ref[g, :, 1, :]
        mag2 = re * re + im * im
        m = jnp.sqrt(mag2 + _EPS)
        carry = s0_ref[...]
        chunks = []
        o = 0
        while o < t:
            L = min(_LC, t - o)
            m_c = m[:, o : o + L]
            u = u_full[:L, :L]
            a = a_full[:, :L]
            states = (
                jnp.dot(m_c, u, preferred_element_type=jnp.float32) + carry * a
            )
            carry = states[:, L - 1 : L]
            chunks.append(states)
            o += L
        states_full = jnp.concatenate(chunks, axis=1)
        r = jax.lax.rsqrt(states_full)
        o_ref[g, :, 0, :] = re * r
        o_ref[g, :, 1, :] = im * r


def kernel(x, init_state):
    b, c, t, f, p = x.shape
    bc = b * c
    xt = jnp.transpose(x, (0, 1, 3, 4, 2)).reshape(bc, f, p, t)
    s0 = init_state.reshape(f, 1)
    u_np, a_np = _scan_mats(_LC)

    gb = 2
    out = pl.pallas_call(
        _eun_kernel,
        out_shape=jax.ShapeDtypeStruct((bc, f, p, t), x.dtype),
        grid=(bc // gb,),
        in_specs=[
            pl.BlockSpec((gb, f, p, t), lambda i: (i, 0, 0, 0)),
            pl.BlockSpec((_LC, _LC), lambda i: (0, 0)),
            pl.BlockSpec((1, _LC), lambda i: (0, 0)),
            pl.BlockSpec((f, 1), lambda i: (0, 0)),
        ],
        out_specs=pl.BlockSpec((gb, f, p, t), lambda i: (i, 0, 0, 0)),
        compiler_params=pltpu.CompilerParams(
            dimension_semantics=("arbitrary",),
            vmem_limit_bytes=48 * 1024 * 1024,
        ),
        name="exp_unit_norm",
    )(xt, jnp.asarray(u_np), jnp.asarray(a_np), s0)

    return jnp.transpose(out.reshape(b, c, f, p, t), (0, 1, 4, 2, 3))
